# Initial kernel scaffold; baseline (speedup 1.0000x reference)
import jax
import jax.numpy as jnp
from jax.experimental import pallas as pl


def kernel(h, edge_index):
    raise NotImplementedError("write your pallas kernel here")


# speedup vs baseline: 4.3974x; 4.3974x over previous
import functools

import jax
import jax.numpy as jnp
from jax import lax
from jax.experimental import pallas as pl
from jax.experimental.pallas import tpu as pltpu
from jax.experimental.pallas import tpu_sc as plsc

N_NODES = 10000
N_EDGES = 320000
D_FEAT = 128

NC = 2
NS = 16
NW = NC * NS

E_PER_W = N_EDGES // NW
CHUNK = 128
NBUF = 4
N_CHUNKS = 80
LAST_OFF = E_PER_W - CHUNK


def _gather_body(h_hbm, src_hbm, out_hbm, idx_bufs, row_bufs, gat_sems, out_sems):
    wid = lax.axis_index("s") * NC + lax.axis_index("c")
    base = wid * E_PER_W

    def chunk_off(j):
        return base + jnp.minimum(j * CHUNK, LAST_OFF)

    def fill(b, j):
        off = chunk_off(j)
        pltpu.sync_copy(src_hbm.at[pl.ds(off, CHUNK)], idx_bufs[b])
        pltpu.async_copy(h_hbm.at[idx_bufs[b]], row_bufs[b], gat_sems[b])

    def drain(b, j):
        off = chunk_off(j)
        pltpu.make_async_copy(h_hbm.at[idx_bufs[b]], row_bufs[b], gat_sems[b]).wait()
        pltpu.async_copy(row_bufs[b], out_hbm.at[pl.ds(off, CHUNK)], out_sems[b])

    for b in range(NBUF):
        fill(b, b)

    def group(gi, carry):
        g = gi * NBUF
        for b in range(NBUF):
            drain(b, g + b)
            pltpu.make_async_copy(
                row_bufs[b], out_hbm.at[pl.ds(chunk_off(g + b), CHUNK)], out_sems[b]
            ).wait()
            fill(b, g + b + NBUF)
        return carry

    lax.fori_loop(0, N_CHUNKS // NBUF - 1, group, 0)

    tail = N_CHUNKS - NBUF
    for b in range(NBUF):
        drain(b, tail + b)
    for b in range(NBUF):
        pltpu.make_async_copy(
            row_bufs[b], out_hbm.at[pl.ds(chunk_off(tail + b), CHUNK)], out_sems[b]
        ).wait()


def _sc_gather(h, src):
    mesh = plsc.VectorSubcoreMesh(
        core_axis_name="c", subcore_axis_name="s", num_cores=NC, num_subcores=NS
    )
    scratch = (
        [pltpu.VMEM((CHUNK,), jnp.int32) for _ in range(NBUF)],
        [pltpu.VMEM((CHUNK, D_FEAT), jnp.float32) for _ in range(NBUF)],
        [pltpu.SemaphoreType.DMA for _ in range(NBUF)],
        [pltpu.SemaphoreType.DMA for _ in range(NBUF)],
    )
    run = pl.kernel(
        _gather_body,
        out_type=jax.ShapeDtypeStruct((N_EDGES, D_FEAT), jnp.float32),
        mesh=mesh,
        scratch_types=scratch,
        name="sc_edge_gather",
    )
    return run(h, src)


@jax.jit
def kernel(h, edge_index):
    src = edge_index[0].astype(jnp.int32)
    return _sc_gather(h, src)
